# Initial kernel scaffold; baseline (speedup 1.0000x reference)
import jax
import jax.numpy as jnp
from jax.experimental import pallas as pl


def kernel(feat, src_trg_edges, pca_w, pca_b, mlp_w, mlp_b):
    raise NotImplementedError("write your pallas kernel here")


# speedup vs baseline: 1.0454x; 1.0454x over previous
import functools

import jax
import jax.numpy as jnp
from jax.experimental import pallas as pl

NCAPS = 8
ROUTIT = 6
NLAYER = 3
D = 128
DD = D // NCAPS


def _dense_relu_body(x_ref, w_ref, b_ref, o_ref):
    o_ref[...] = jax.nn.relu(
        jnp.dot(x_ref[...], w_ref[...], preferred_element_type=jnp.float32)
        + b_ref[...]
    )


def _dense_softmax_body(x_ref, w_ref, b_ref, o_ref):
    logits = (
        jnp.dot(x_ref[...], w_ref[...], preferred_element_type=jnp.float32)
        + b_ref[...]
    )
    o_ref[...] = jax.nn.softmax(logits, axis=-1)


def _capsule_block_diag():
    i = jnp.arange(D)
    return (i[:, None] // DD == i[None, :] // DD).astype(jnp.float32)


def _norm_body(x_ref, b_ref, o_ref):
    x = x_ref[...]
    nrm2 = jnp.dot(x * x, b_ref[...], preferred_element_type=jnp.float32)
    o_ref[...] = x / jnp.maximum(jnp.sqrt(nrm2), 1e-12)


def _addnorm_body(x_ref, a_ref, b_ref, o_ref):
    x = x_ref[...] + a_ref[...]
    nrm2 = jnp.dot(x * x, b_ref[...], preferred_element_type=jnp.float32)
    o_ref[...] = x / jnp.maximum(jnp.sqrt(nrm2), 1e-12)


def _tc_dense_relu(x, w, b):
    n = x.shape[0]
    return pl.pallas_call(
        _dense_relu_body,
        out_shape=jax.ShapeDtypeStruct((n, w.shape[1]), jnp.float32),
    )(x, w, b)


def _tc_dense_softmax(x, w, b):
    n = x.shape[0]
    return pl.pallas_call(
        _dense_softmax_body,
        out_shape=jax.ShapeDtypeStruct((n, w.shape[1]), jnp.float32),
    )(x, w, b)


def _tc_norm(x):
    n = x.shape[0]
    blk = 2000
    return pl.pallas_call(
        _norm_body,
        grid=(n // blk,),
        in_specs=[
            pl.BlockSpec((blk, D), lambda i: (i, 0)),
            pl.BlockSpec((D, D), lambda i: (0, 0)),
        ],
        out_specs=pl.BlockSpec((blk, D), lambda i: (i, 0)),
        out_shape=jax.ShapeDtypeStruct(x.shape, jnp.float32),
    )(x, _capsule_block_diag())


def _tc_addnorm(x, agg):
    n = x.shape[0]
    blk = 2000
    return pl.pallas_call(
        _addnorm_body,
        grid=(n // blk,),
        in_specs=[
            pl.BlockSpec((blk, D), lambda i: (i, 0)),
            pl.BlockSpec((blk, D), lambda i: (i, 0)),
            pl.BlockSpec((D, D), lambda i: (0, 0)),
        ],
        out_specs=pl.BlockSpec((blk, D), lambda i: (i, 0)),
        out_shape=jax.ShapeDtypeStruct(x.shape, jnp.float32),
    )(x, agg, _capsule_block_diag())


def _routing_layer(x, src, trg):
    n, d = x.shape
    m = src.shape[0]
    x = _tc_norm(x)
    z = x[src].reshape(m, NCAPS, DD)
    c = x
    for _ in range(ROUTIT):
        p = jnp.sum(z * c[trg].reshape(m, NCAPS, DD), axis=2)
        p = jax.nn.softmax(p, axis=1)
        weighted = (z * p[:, :, None]).reshape(m, d)
        agg = jax.ops.segment_sum(weighted, trg, num_segments=n)
        c = _tc_addnorm(x, agg)
    return c


def kernel(feat, src_trg_edges, pca_w, pca_b, mlp_w, mlp_b):
    x = _tc_dense_relu(feat, pca_w, pca_b)
    src = src_trg_edges[0]
    trg = src_trg_edges[1]
    for _ in range(NLAYER):
        x = _routing_layer(x, src, trg)
    return _tc_dense_softmax(x, mlp_w, mlp_b)
